# Initial kernel scaffold; baseline (speedup 1.0000x reference)
import jax
import jax.numpy as jnp
from jax.experimental import pallas as pl


def kernel(x, edge_index, W_l1, W_r1, b1, gamma1, beta1, W_l2, W_r2, b2, gamma2, beta2, W_fc, b_fc):
    raise NotImplementedError("write your pallas kernel here")


# speedup vs baseline: 5.6830x; 5.6830x over previous
import functools

import jax
import jax.numpy as jnp
from jax import lax
from jax.experimental import pallas as pl
from jax.experimental.pallas import tpu as pltpu
from jax.experimental.pallas import tpu_sc as plsc

_NC = 2
_NS = 16
_NW = _NC * _NS


@functools.cache
def _make_seg_sum(E, NPAD, D, CH, IB, with_deg=False):
    e_per_w = E // _NW
    n_ch = e_per_w // CH
    n_ib = n_ch // IB
    assert n_ch * CH == e_per_w and CH % 8 == 0 and CH <= 128
    assert n_ib * IB == n_ch
    rows_per_tile = NPAD // _NS
    n_piece = rows_per_tile // CH
    assert n_piece * CH == rows_per_tile

    mesh = plsc.VectorSubcoreMesh(
        core_axis_name="c", subcore_axis_name="s",
        num_cores=_NC, num_subcores=_NS)

    out_type = jax.ShapeDtypeStruct((_NC, NPAD, D), jnp.float32)
    scratch = [
        pltpu.VMEM((IB, CH), jnp.int32),
        pltpu.VMEM((IB, CH), jnp.int32),
        pltpu.VMEM((CH,), jnp.int32),
        pltpu.VMEM((CH,), jnp.int32),
        pltpu.VMEM((CH, D), jnp.float32),
        pltpu.VMEM_SHARED((NPAD, D), jnp.float32),
        pltpu.SemaphoreType.DMA,
    ]
    def body(src_hbm, dst_hbm, x_hbm, msg_out,
             src_v, dst_v, src_c, dst_c, rows_v, acc_sh, sem):
        cid = lax.axis_index("c")
        sid = lax.axis_index("s")
        wid = sid * _NC + cid
        r0 = sid * rows_per_tile

        zv = jnp.zeros((16,), jnp.float32)

        def zrow(r, carry):
            for k in range(D // 16):
                rows_v[r, pl.ds(k * 16, 16)] = zv
            return carry

        lax.fori_loop(0, CH, zrow, 0)
        for p in range(n_piece):
            pltpu.sync_copy(rows_v, acc_sh.at[pl.ds(r0 + p * CH, CH)])
        plsc.subcore_barrier()

        def block(ib, carry):
            pltpu.sync_copy(src_hbm.at[wid, ib], src_v)
            pltpu.sync_copy(dst_hbm.at[wid, ib], dst_v)

            def chunk(c, carry2):
                for j in range(CH // 16):
                    sl = pl.ds(j * 16, 16)
                    src_c[sl] = src_v[c, sl]
                    dst_c[sl] = dst_v[c, sl]
                pltpu.async_copy(x_hbm.at[src_c], rows_v, sem).wait()
                pltpu.sync_copy(rows_v, acc_sh.at[dst_c], add=True)
                return carry2

            return lax.fori_loop(0, IB, chunk, carry)

        lax.fori_loop(0, n_ib, block, 0)
        plsc.subcore_barrier()

        for p in range(n_piece):
            sl = pl.ds(r0 + p * CH, CH)
            pltpu.sync_copy(acc_sh.at[sl], rows_v)
            pltpu.sync_copy(rows_v, msg_out.at[cid, sl])

    return pl.kernel(body, out_type=out_type, mesh=mesh,
                     scratch_types=scratch)


def _seg_sum(src, dst, x_pad):
    _, n_ib, IB, CH = src.shape
    E = _NW * n_ib * IB * CH
    NPAD, D = x_pad.shape
    fn = _make_seg_sum(E, NPAD, D, CH, IB)
    return fn(src, dst, x_pad)


@functools.cache
def _make_deg(E, NPAD, D, CH, IB):
    e_per_w = E // _NW
    n_ch = e_per_w // CH
    n_ib = n_ch // IB
    rows_per_tile = NPAD // _NS
    n_piece = rows_per_tile // CH

    mesh = plsc.VectorSubcoreMesh(
        core_axis_name="c", subcore_axis_name="s",
        num_cores=_NC, num_subcores=_NS)

    out_type = jax.ShapeDtypeStruct((_NC, NPAD, D), jnp.float32)
    scratch = [
        pltpu.VMEM((IB, CH), jnp.int32),
        pltpu.VMEM((CH,), jnp.int32),
        pltpu.VMEM((CH, D), jnp.float32),
        pltpu.VMEM_SHARED((NPAD, D), jnp.float32),
    ]

    def body(dst_hbm, deg_out, dst_v, dst_c, ones_v, acc_sh):
        cid = lax.axis_index("c")
        sid = lax.axis_index("s")
        wid = sid * _NC + cid
        r0 = sid * rows_per_tile

        def fill(val):
            v = jnp.full((16,), val, jnp.float32)

            def frow(r, carry):
                for k in range(D // 16):
                    ones_v[r, pl.ds(k * 16, 16)] = v
                return carry

            lax.fori_loop(0, CH, frow, 0)

        fill(0.0)
        for p in range(n_piece):
            pltpu.sync_copy(ones_v, acc_sh.at[pl.ds(r0 + p * CH, CH)])
        fill(1.0)
        plsc.subcore_barrier()

        def block(ib, carry):
            pltpu.sync_copy(dst_hbm.at[wid, ib], dst_v)

            def chunk(c, carry2):
                for j in range(CH // 16):
                    sl = pl.ds(j * 16, 16)
                    dst_c[sl] = dst_v[c, sl]
                pltpu.sync_copy(ones_v, acc_sh.at[dst_c], add=True)
                return carry2

            return lax.fori_loop(0, IB, chunk, carry)

        lax.fori_loop(0, n_ib, block, 0)
        plsc.subcore_barrier()

        for p in range(n_piece):
            sl = pl.ds(r0 + p * CH, CH)
            pltpu.sync_copy(acc_sh.at[sl], ones_v)
            pltpu.sync_copy(ones_v, deg_out.at[cid, sl])

    return pl.kernel(body, out_type=out_type, mesh=mesh,
                     scratch_types=scratch)


def _deg_hist(dst, NPAD, D):
    _, n_ib, IB, CH = dst.shape
    E = _NW * n_ib * IB * CH
    return _make_deg(E, NPAD, D, CH, IB)(dst)


def _dense_layer(msg_parts, deg_parts, xin, W_l, W_r, b, n_valid, blk=512):
    NPAD, D = xin.shape
    Dm = msg_parts.shape[2]
    nblk = NPAD // blk

    def body(msg_ref, deg_ref, x_ref, wl_ref, wr_ref, b_ref,
             z_ref, stats_ref, acc_ref):
        i = pl.program_id(0)
        msg = msg_ref[0] + msg_ref[1]
        deg = deg_ref[0, :, 0:1] + deg_ref[1, :, 0:1]
        agg = msg / jnp.maximum(deg, 1.0)
        z = (jnp.dot(agg, wl_ref[...], preferred_element_type=jnp.float32)
             + jnp.dot(x_ref[...], wr_ref[...], preferred_element_type=jnp.float32)
             + b_ref[...])
        rid = i * blk + lax.broadcasted_iota(jnp.int32, (blk, 1), 0)
        z = jnp.where(rid < n_valid, z, 0.0)
        z_ref[...] = z

        @pl.when(i == 0)
        def _():
            acc_ref[...] = jnp.zeros_like(acc_ref)

        acc_ref[0:1] += jnp.sum(z, axis=0, keepdims=True)
        acc_ref[1:2] += jnp.sum(z * z, axis=0, keepdims=True)

        @pl.when(i == nblk - 1)
        def _():
            stats_ref[...] = acc_ref[...]

    return pl.pallas_call(
        body,
        grid=(nblk,),
        in_specs=[
            pl.BlockSpec((2, blk, Dm), lambda i: (0, i, 0)),
            pl.BlockSpec((2, blk, 8), lambda i: (0, i, 0)),
            pl.BlockSpec((blk, D), lambda i: (i, 0)),
            pl.BlockSpec((D, D), lambda i: (0, 0)),
            pl.BlockSpec((D, D), lambda i: (0, 0)),
            pl.BlockSpec((1, D), lambda i: (0, 0)),
        ],
        out_specs=[
            pl.BlockSpec((blk, D), lambda i: (i, 0)),
            pl.BlockSpec((8, D), lambda i: (0, 0)),
        ],
        out_shape=[
            jax.ShapeDtypeStruct((NPAD, D), jnp.float32),
            jax.ShapeDtypeStruct((8, D), jnp.float32),
        ],
        scratch_shapes=[pltpu.VMEM((8, D), jnp.float32)],
    )(msg_parts, deg_parts, xin, W_l, W_r, b.reshape(1, D))


def _bn_relu(z, stats, gamma, beta, n_valid, blk=512):
    NPAD, D = z.shape
    nblk = NPAD // blk
    inv_n = 1.0 / n_valid

    def body(z_ref, stats_ref, g_ref, be_ref, out_ref):
        mean = stats_ref[0:1] * inv_n
        var = stats_ref[1:2] * inv_n - mean * mean
        inv = lax.rsqrt(var + 1e-5)
        out_ref[...] = jnp.maximum(
            (z_ref[...] - mean) * inv * g_ref[...] + be_ref[...], 0.0)

    return pl.pallas_call(
        body,
        grid=(nblk,),
        in_specs=[
            pl.BlockSpec((blk, D), lambda i: (i, 0)),
            pl.BlockSpec((8, D), lambda i: (0, 0)),
            pl.BlockSpec((1, D), lambda i: (0, 0)),
            pl.BlockSpec((1, D), lambda i: (0, 0)),
        ],
        out_specs=pl.BlockSpec((blk, D), lambda i: (i, 0)),
        out_shape=jax.ShapeDtypeStruct((NPAD, D), jnp.float32),
    )(z, stats, gamma.reshape(1, D), beta.reshape(1, D))


def _bn_relu_fc_softmax(z, stats, gamma, beta, W_fc_pad, b_fc_pad,
                        n_valid, blk=512):
    NPAD, D = z.shape
    nblk = NPAD // blk
    inv_n = 1.0 / n_valid

    def body(z_ref, stats_ref, g_ref, be_ref, wfc_ref, bfc_ref,
             x0_ref, x1_ref):
        mean = stats_ref[0:1] * inv_n
        var = stats_ref[1:2] * inv_n - mean * mean
        inv = lax.rsqrt(var + 1e-5)
        x0 = jnp.maximum(
            (z_ref[...] - mean) * inv * g_ref[...] + be_ref[...], 0.0)
        x0_ref[...] = x0
        logits = (jnp.dot(x0, wfc_ref[...], preferred_element_type=jnp.float32)
                  + bfc_ref[...])
        m = jnp.max(logits, axis=1, keepdims=True)
        e = jnp.exp(logits - m)
        x1_ref[...] = e / jnp.sum(e, axis=1, keepdims=True)

    return pl.pallas_call(
        body,
        grid=(nblk,),
        in_specs=[
            pl.BlockSpec((blk, D), lambda i: (i, 0)),
            pl.BlockSpec((8, D), lambda i: (0, 0)),
            pl.BlockSpec((1, D), lambda i: (0, 0)),
            pl.BlockSpec((1, D), lambda i: (0, 0)),
            pl.BlockSpec((D, D), lambda i: (0, 0)),
            pl.BlockSpec((1, D), lambda i: (0, 0)),
        ],
        out_specs=[
            pl.BlockSpec((blk, D), lambda i: (i, 0)),
            pl.BlockSpec((blk, D), lambda i: (i, 0)),
        ],
        out_shape=[
            jax.ShapeDtypeStruct((NPAD, D), jnp.float32),
            jax.ShapeDtypeStruct((NPAD, D), jnp.float32),
        ],
    )(z, stats, gamma.reshape(1, D), beta.reshape(1, D), W_fc_pad, b_fc_pad)


def kernel(x, edge_index, W_l1, W_r1, b1, gamma1, beta1,
           W_l2, W_r2, b2, gamma2, beta2, W_fc, b_fc):
    N, D = x.shape
    E = edge_index.shape[1]
    NC_OUT = W_fc.shape[1]
    NPAD = ((N + 1023) // 1024) * 1024
    CH = 80
    e_per_w = E // _NW
    n_ch = e_per_w // CH

    IB = 25
    src = edge_index[0].reshape(_NW, n_ch // IB, IB, CH)
    dst = edge_index[1].reshape(_NW, n_ch // IB, IB, CH)
    x_pad = jnp.zeros((NPAD, D), jnp.float32).at[:N].set(x)

    deg_parts = _deg_hist(dst, NPAD, D)
    deg = lax.slice(deg_parts, (0, 0, 0), (2, NPAD, 8))
    msg1 = _seg_sum(src, dst, x_pad)
    z1, stats1 = _dense_layer(msg1, deg, x_pad, W_l1, W_r1, b1, N)
    h1 = _bn_relu(z1, stats1, gamma1, beta1, N)

    msg2 = _seg_sum(src, dst, h1)
    z2, stats2 = _dense_layer(msg2, deg, h1, W_l2, W_r2, b2, N)

    W_fc_pad = jnp.zeros((D, D), jnp.float32).at[:, :NC_OUT].set(W_fc)
    b_fc_pad = jnp.full((1, D), -1e30, jnp.float32).at[0, :NC_OUT].set(b_fc)
    x0_full, x1_full = _bn_relu_fc_softmax(
        z2, stats2, gamma2, beta2, W_fc_pad, b_fc_pad, N)

    return (x0_full[:N], x1_full[:N, :NC_OUT])


# speedup vs baseline: 5.7758x; 1.0163x over previous
import functools

import jax
import jax.numpy as jnp
from jax import lax
from jax.experimental import pallas as pl
from jax.experimental.pallas import tpu as pltpu
from jax.experimental.pallas import tpu_sc as plsc

_NC = 2
_NS = 16
_NW = _NC * _NS


@functools.cache
def _make_seg_sum(E, NPAD, D, CH, IB, with_deg=False):
    e_per_w = E // _NW
    n_ch = e_per_w // CH
    n_ib = n_ch // IB
    assert n_ch * CH == e_per_w and CH % 8 == 0 and CH <= 128
    assert n_ib * IB == n_ch
    rows_per_tile = NPAD // _NS
    n_piece = rows_per_tile // CH
    assert n_piece * CH == rows_per_tile

    mesh = plsc.VectorSubcoreMesh(
        core_axis_name="c", subcore_axis_name="s",
        num_cores=_NC, num_subcores=_NS)

    out_type = jax.ShapeDtypeStruct((_NC, NPAD, D), jnp.float32)
    scratch = [
        pltpu.VMEM((IB, CH), jnp.int32),
        pltpu.VMEM((IB, CH), jnp.int32),
        pltpu.VMEM((CH,), jnp.int32),
        pltpu.VMEM((CH,), jnp.int32),
        pltpu.VMEM((CH, D), jnp.float32),
        pltpu.VMEM_SHARED((NPAD, D), jnp.float32),
        pltpu.SemaphoreType.DMA,
    ]
    def body(src_hbm, dst_hbm, x_hbm, msg_out,
             src_v, dst_v, src_c, dst_c, rows_v, acc_sh, sem):
        cid = lax.axis_index("c")
        sid = lax.axis_index("s")
        wid = sid * _NC + cid
        r0 = sid * rows_per_tile

        zv = jnp.zeros((16,), jnp.float32)

        def zrow(r, carry):
            for k in range(D // 16):
                rows_v[r, pl.ds(k * 16, 16)] = zv
            return carry

        lax.fori_loop(0, CH, zrow, 0)
        for p in range(n_piece):
            pltpu.sync_copy(rows_v, acc_sh.at[pl.ds(r0 + p * CH, CH)])
        plsc.subcore_barrier()

        def block(ib, carry):
            pltpu.sync_copy(src_hbm.at[wid, ib], src_v)
            pltpu.sync_copy(dst_hbm.at[wid, ib], dst_v)

            def chunk(c, carry2):
                pltpu.async_copy(x_hbm.at[src_v.at[c]], rows_v, sem).wait()
                pltpu.sync_copy(rows_v, acc_sh.at[dst_v.at[c]], add=True)
                return carry2

            return lax.fori_loop(0, IB, chunk, carry)

        lax.fori_loop(0, n_ib, block, 0)
        plsc.subcore_barrier()

        for p in range(n_piece):
            sl = pl.ds(r0 + p * CH, CH)
            pltpu.sync_copy(acc_sh.at[sl], rows_v)
            pltpu.sync_copy(rows_v, msg_out.at[cid, sl])

    return pl.kernel(body, out_type=out_type, mesh=mesh,
                     scratch_types=scratch)


def _seg_sum(src, dst, x_pad):
    _, n_ib, IB, CH = src.shape
    E = _NW * n_ib * IB * CH
    NPAD, D = x_pad.shape
    fn = _make_seg_sum(E, NPAD, D, CH, IB)
    return fn(src, dst, x_pad)


@functools.cache
def _make_deg(E, NPAD, D, CH, IB):
    e_per_w = E // _NW
    n_ch = e_per_w // CH
    n_ib = n_ch // IB
    rows_per_tile = NPAD // _NS
    n_piece = rows_per_tile // CH

    mesh = plsc.VectorSubcoreMesh(
        core_axis_name="c", subcore_axis_name="s",
        num_cores=_NC, num_subcores=_NS)

    out_type = jax.ShapeDtypeStruct((_NC, NPAD, D), jnp.float32)
    scratch = [
        pltpu.VMEM((IB, CH), jnp.int32),
        pltpu.VMEM((CH,), jnp.int32),
        pltpu.VMEM((CH, D), jnp.float32),
        pltpu.VMEM_SHARED((NPAD, D), jnp.float32),
    ]

    def body(dst_hbm, deg_out, dst_v, dst_c, ones_v, acc_sh):
        cid = lax.axis_index("c")
        sid = lax.axis_index("s")
        wid = sid * _NC + cid
        r0 = sid * rows_per_tile

        def fill(val):
            v = jnp.full((16,), val, jnp.float32)

            def frow(r, carry):
                for k in range(D // 16):
                    ones_v[r, pl.ds(k * 16, 16)] = v
                return carry

            lax.fori_loop(0, CH, frow, 0)

        fill(0.0)
        for p in range(n_piece):
            pltpu.sync_copy(ones_v, acc_sh.at[pl.ds(r0 + p * CH, CH)])
        fill(1.0)
        plsc.subcore_barrier()

        def block(ib, carry):
            pltpu.sync_copy(dst_hbm.at[wid, ib], dst_v)

            def chunk(c, carry2):
                pltpu.sync_copy(ones_v, acc_sh.at[dst_v.at[c]], add=True)
                return carry2

            return lax.fori_loop(0, IB, chunk, carry)

        lax.fori_loop(0, n_ib, block, 0)
        plsc.subcore_barrier()

        for p in range(n_piece):
            sl = pl.ds(r0 + p * CH, CH)
            pltpu.sync_copy(acc_sh.at[sl], ones_v)
            pltpu.sync_copy(ones_v, deg_out.at[cid, sl])

    return pl.kernel(body, out_type=out_type, mesh=mesh,
                     scratch_types=scratch)


def _deg_hist(dst, NPAD, D):
    _, n_ib, IB, CH = dst.shape
    E = _NW * n_ib * IB * CH
    return _make_deg(E, NPAD, D, CH, IB)(dst)


def _dense_layer(msg_parts, deg_parts, xin, W_l, W_r, b, n_valid, blk=512):
    NPAD, D = xin.shape
    Dm = msg_parts.shape[2]
    nblk = NPAD // blk

    def body(msg_ref, deg_ref, x_ref, wl_ref, wr_ref, b_ref,
             z_ref, stats_ref, acc_ref):
        i = pl.program_id(0)
        msg = msg_ref[0] + msg_ref[1]
        deg = deg_ref[0, :, 0:1] + deg_ref[1, :, 0:1]
        agg = msg / jnp.maximum(deg, 1.0)
        z = (jnp.dot(agg, wl_ref[...], preferred_element_type=jnp.float32)
             + jnp.dot(x_ref[...], wr_ref[...], preferred_element_type=jnp.float32)
             + b_ref[...])
        rid = i * blk + lax.broadcasted_iota(jnp.int32, (blk, 1), 0)
        z = jnp.where(rid < n_valid, z, 0.0)
        z_ref[...] = z

        @pl.when(i == 0)
        def _():
            acc_ref[...] = jnp.zeros_like(acc_ref)

        acc_ref[0:1] += jnp.sum(z, axis=0, keepdims=True)
        acc_ref[1:2] += jnp.sum(z * z, axis=0, keepdims=True)

        @pl.when(i == nblk - 1)
        def _():
            stats_ref[...] = acc_ref[...]

    return pl.pallas_call(
        body,
        grid=(nblk,),
        in_specs=[
            pl.BlockSpec((2, blk, Dm), lambda i: (0, i, 0)),
            pl.BlockSpec((2, blk, 8), lambda i: (0, i, 0)),
            pl.BlockSpec((blk, D), lambda i: (i, 0)),
            pl.BlockSpec((D, D), lambda i: (0, 0)),
            pl.BlockSpec((D, D), lambda i: (0, 0)),
            pl.BlockSpec((1, D), lambda i: (0, 0)),
        ],
        out_specs=[
            pl.BlockSpec((blk, D), lambda i: (i, 0)),
            pl.BlockSpec((8, D), lambda i: (0, 0)),
        ],
        out_shape=[
            jax.ShapeDtypeStruct((NPAD, D), jnp.float32),
            jax.ShapeDtypeStruct((8, D), jnp.float32),
        ],
        scratch_shapes=[pltpu.VMEM((8, D), jnp.float32)],
    )(msg_parts, deg_parts, xin, W_l, W_r, b.reshape(1, D))


def _bn_relu(z, stats, gamma, beta, n_valid, blk=512):
    NPAD, D = z.shape
    nblk = NPAD // blk
    inv_n = 1.0 / n_valid

    def body(z_ref, stats_ref, g_ref, be_ref, out_ref):
        mean = stats_ref[0:1] * inv_n
        var = stats_ref[1:2] * inv_n - mean * mean
        inv = lax.rsqrt(var + 1e-5)
        out_ref[...] = jnp.maximum(
            (z_ref[...] - mean) * inv * g_ref[...] + be_ref[...], 0.0)

    return pl.pallas_call(
        body,
        grid=(nblk,),
        in_specs=[
            pl.BlockSpec((blk, D), lambda i: (i, 0)),
            pl.BlockSpec((8, D), lambda i: (0, 0)),
            pl.BlockSpec((1, D), lambda i: (0, 0)),
            pl.BlockSpec((1, D), lambda i: (0, 0)),
        ],
        out_specs=pl.BlockSpec((blk, D), lambda i: (i, 0)),
        out_shape=jax.ShapeDtypeStruct((NPAD, D), jnp.float32),
    )(z, stats, gamma.reshape(1, D), beta.reshape(1, D))


def _bn_relu_fc_softmax(z, stats, gamma, beta, W_fc_pad, b_fc_pad,
                        n_valid, blk=512):
    NPAD, D = z.shape
    nblk = NPAD // blk
    inv_n = 1.0 / n_valid

    def body(z_ref, stats_ref, g_ref, be_ref, wfc_ref, bfc_ref,
             x0_ref, x1_ref):
        mean = stats_ref[0:1] * inv_n
        var = stats_ref[1:2] * inv_n - mean * mean
        inv = lax.rsqrt(var + 1e-5)
        x0 = jnp.maximum(
            (z_ref[...] - mean) * inv * g_ref[...] + be_ref[...], 0.0)
        x0_ref[...] = x0
        logits = (jnp.dot(x0, wfc_ref[...], preferred_element_type=jnp.float32)
                  + bfc_ref[...])
        m = jnp.max(logits, axis=1, keepdims=True)
        e = jnp.exp(logits - m)
        x1_ref[...] = e / jnp.sum(e, axis=1, keepdims=True)

    return pl.pallas_call(
        body,
        grid=(nblk,),
        in_specs=[
            pl.BlockSpec((blk, D), lambda i: (i, 0)),
            pl.BlockSpec((8, D), lambda i: (0, 0)),
            pl.BlockSpec((1, D), lambda i: (0, 0)),
            pl.BlockSpec((1, D), lambda i: (0, 0)),
            pl.BlockSpec((D, D), lambda i: (0, 0)),
            pl.BlockSpec((1, D), lambda i: (0, 0)),
        ],
        out_specs=[
            pl.BlockSpec((blk, D), lambda i: (i, 0)),
            pl.BlockSpec((blk, D), lambda i: (i, 0)),
        ],
        out_shape=[
            jax.ShapeDtypeStruct((NPAD, D), jnp.float32),
            jax.ShapeDtypeStruct((NPAD, D), jnp.float32),
        ],
    )(z, stats, gamma.reshape(1, D), beta.reshape(1, D), W_fc_pad, b_fc_pad)


def kernel(x, edge_index, W_l1, W_r1, b1, gamma1, beta1,
           W_l2, W_r2, b2, gamma2, beta2, W_fc, b_fc):
    N, D = x.shape
    E = edge_index.shape[1]
    NC_OUT = W_fc.shape[1]
    NPAD = ((N + 1023) // 1024) * 1024
    CH = 80
    e_per_w = E // _NW
    n_ch = e_per_w // CH

    IB = 25
    src = edge_index[0].reshape(_NW, n_ch // IB, IB, CH)
    dst = edge_index[1].reshape(_NW, n_ch // IB, IB, CH)
    x_pad = jnp.zeros((NPAD, D), jnp.float32).at[:N].set(x)

    deg_parts = _deg_hist(dst, NPAD, D)
    deg = lax.slice(deg_parts, (0, 0, 0), (2, NPAD, 8))
    msg1 = _seg_sum(src, dst, x_pad)
    z1, stats1 = _dense_layer(msg1, deg, x_pad, W_l1, W_r1, b1, N)
    h1 = _bn_relu(z1, stats1, gamma1, beta1, N)

    msg2 = _seg_sum(src, dst, h1)
    z2, stats2 = _dense_layer(msg2, deg, h1, W_l2, W_r2, b2, N)

    W_fc_pad = jnp.zeros((D, D), jnp.float32).at[:, :NC_OUT].set(W_fc)
    b_fc_pad = jnp.full((1, D), -1e30, jnp.float32).at[0, :NC_OUT].set(b_fc)
    x0_full, x1_full = _bn_relu_fc_softmax(
        z2, stats2, gamma2, beta2, W_fc_pad, b_fc_pad, N)

    return (x0_full[:N], x1_full[:N, :NC_OUT])
